# baseline (speedup 1.0000x reference)
import functools

import jax
from jax import lax
import jax.numpy as jnp
from jax.experimental import pallas as pl
from jax.experimental.pallas import tpu as pltpu
from jax.experimental.pallas import tpu_sc as plsc

_EPS = 1e-5
_TC_BLOCK = 8192
_GATHER_WINDOW = 256
_TR_COLS = 16384


def _tc_pair_table_bulk(emb_t, nblocks):
    d = emb_t.shape[0]
    c = _TR_COLS

    def tr_kernel(x_ref, o_ref):
        ii = jax.lax.broadcasted_iota(jnp.int32, (d, 2 * d), 0)
        jj = jax.lax.broadcasted_iota(jnp.int32, (d, 2 * d), 1)
        e1 = (ii == jj).astype(jnp.float32)
        e2 = (ii + d == jj).astype(jnp.float32)
        dn = (((0,), (0,)), ((), ()))
        left = jax.lax.dot_general(
            x_ref[:, : c // 2], e1, dn, preferred_element_type=jnp.float32
        )
        right = jax.lax.dot_general(
            x_ref[:, c // 2 :], e2, dn, preferred_element_type=jnp.float32
        )
        o_ref[...] = left + right

    return pl.pallas_call(
        tr_kernel,
        grid=(nblocks,),
        in_specs=[pl.BlockSpec((d, c), lambda i: (0, i))],
        out_specs=pl.BlockSpec((c // 2, 2 * d), lambda i: (i, 0)),
        out_shape=jax.ShapeDtypeStruct((nblocks * c // 2, 2 * d), jnp.float32),
    )(emb_t)


def _tc_pair_table_tail(tail_t):
    d, m = tail_t.shape

    def tr_kernel(x_ref, o_ref):
        t = jnp.swapaxes(x_ref[...], 0, 1)
        o_ref[:, :d] = t[: m // 2]
        o_ref[:, d:] = t[m // 2 :]

    return pl.pallas_call(
        tr_kernel,
        in_specs=[pl.BlockSpec((d, m), lambda: (0, 0))],
        out_specs=pl.BlockSpec((m // 2, 2 * d), lambda: (0, 0)),
        out_shape=jax.ShapeDtypeStruct((m // 2, 2 * d), jnp.float32),
    )(tail_t)


def _sc_gather(bulk, tail, idx_bulk, idx_tail):
    n = idx_bulk.shape[0]
    d = bulk.shape[1]
    ib2 = idx_bulk.reshape(1, n)
    it2 = idx_tail.reshape(1, n)
    mesh = plsc.VectorSubcoreMesh(core_axis_name="c", subcore_axis_name="s")
    w = _GATHER_WINDOW

    @functools.partial(
        pl.kernel,
        mesh=mesh,
        out_type=jax.ShapeDtypeStruct((n, d), bulk.dtype),
    )
    def gather_kernel(bulk_hbm, tail_hbm, ib_hbm, it_hbm, out_hbm):
        def body(ib_vmem, it_vmem, o_vmem):
            pltpu.sync_copy(
                bulk_hbm.at[plsc.Indices(ib_vmem.at[0], ignored_value=-1)],
                o_vmem,
            )
            pltpu.sync_copy(
                tail_hbm.at[plsc.Indices(it_vmem.at[0], ignored_value=-1)],
                o_vmem,
            )

        pltpu.emit_pipeline(
            body,
            grid=(n // w,),
            in_specs=[
                pl.BlockSpec((1, w), index_map=lambda i: (0, i)),
                pl.BlockSpec((1, w), index_map=lambda i: (0, i)),
            ],
            out_specs=[pl.BlockSpec((w, d), index_map=lambda i: (i, 0))],
            core_axis_name=("c", "s"),
            dimension_semantics=(pltpu.PARALLEL,),
        )(ib_hbm, it_hbm, out_hbm)

    return gather_kernel(bulk, tail, ib2, it2)


def _tc_mlp_ln(pairs, sel, W, b, gamma, beta):
    n = pairs.shape[0]
    d = W.shape[0]
    h = W.shape[1]
    b2 = b.reshape(1, h)
    g2 = gamma.reshape(1, h)
    be2 = beta.reshape(1, h)

    def mlp_kernel(pair_ref, sel_ref, w_ref, b_ref, g_ref, be_ref, o_ref):
        p = sel_ref[...]
        x = pair_ref[:, :d] * (1.0 - p) + pair_ref[:, d:] * p
        acc = jnp.dot(x, w_ref[...], preferred_element_type=jnp.float32)
        acc = jnp.maximum(acc + b_ref[...], 0.0)
        mean = jnp.mean(acc, axis=-1, keepdims=True)
        var = jnp.mean((acc - mean) ** 2, axis=-1, keepdims=True)
        xhat = (acc - mean) * jax.lax.rsqrt(var + _EPS)
        o_ref[...] = xhat * g_ref[...] + be_ref[...]

    blk = min(_TC_BLOCK, n)
    return pl.pallas_call(
        mlp_kernel,
        grid=(n // blk,),
        in_specs=[
            pl.BlockSpec((blk, 2 * d), lambda i: (i, 0)),
            pl.BlockSpec((blk, 1), lambda i: (i, 0)),
            pl.BlockSpec((d, h), lambda i: (0, 0)),
            pl.BlockSpec((1, h), lambda i: (0, 0)),
            pl.BlockSpec((1, h), lambda i: (0, 0)),
            pl.BlockSpec((1, h), lambda i: (0, 0)),
        ],
        out_specs=pl.BlockSpec((blk, h), lambda i: (i, 0)),
        out_shape=jax.ShapeDtypeStruct((n, h), jnp.float32),
    )(pairs, sel, W, b2, g2, be2)


def kernel(user_input, emb, W, b, gamma, beta):
    n_rows, d = emb.shape
    c = _TR_COLS
    nblocks = n_rows // c
    bulk_rows = nblocks * c
    emb_t = emb.T
    bulk_pt = _tc_pair_table_bulk(emb_t, nblocks)
    tail_pt = _tc_pair_table_tail(
        lax.slice(emb_t, (0, bulk_rows), (d, n_rows))
    )

    r = user_input
    tail = r >= bulk_rows
    off = r % c
    bulk_prow = (r // c) * (c // 2) + (off % (c // 2))
    bulk_sel = off // (c // 2)
    to = r - bulk_rows
    tail_half = (n_rows - bulk_rows) // 2
    tail_prow = to % tail_half
    tail_sel = to // tail_half
    sel = jnp.where(tail, tail_sel, bulk_sel).astype(jnp.float32).reshape(-1, 1)
    idx_bulk = jnp.where(tail, -1, bulk_prow).astype(jnp.int32)
    idx_tail = jnp.where(tail, tail_prow, -1).astype(jnp.int32)

    gathered = _sc_gather(bulk_pt, tail_pt, idx_bulk, idx_tail)
    return _tc_mlp_ln(gathered, sel, W, b, gamma, beta)


# speedup vs baseline: 1.0738x; 1.0738x over previous
import functools

import jax
from jax import lax
import jax.numpy as jnp
from jax.experimental import pallas as pl
from jax.experimental.pallas import tpu as pltpu
from jax.experimental.pallas import tpu_sc as plsc

_EPS = 1e-5
_TC_BLOCK = 8192
_GATHER_WINDOW = 256
_TR_COLS = 32768


def _tc_pair_table_bulk(emb_t, nblocks):
    d = emb_t.shape[0]
    c = _TR_COLS

    def tr_kernel(x_ref, o_ref):
        ii = jax.lax.broadcasted_iota(jnp.int32, (d, 2 * d), 0)
        jj = jax.lax.broadcasted_iota(jnp.int32, (d, 2 * d), 1)
        e1 = (ii == jj).astype(jnp.float32)
        e2 = (ii + d == jj).astype(jnp.float32)
        dn = (((0,), (0,)), ((), ()))
        left = jax.lax.dot_general(
            x_ref[:, : c // 2], e1, dn, preferred_element_type=jnp.float32
        )
        right = jax.lax.dot_general(
            x_ref[:, c // 2 :], e2, dn, preferred_element_type=jnp.float32
        )
        o_ref[...] = left + right

    return pl.pallas_call(
        tr_kernel,
        grid=(nblocks,),
        in_specs=[pl.BlockSpec((d, c), lambda i: (0, i))],
        out_specs=pl.BlockSpec((c // 2, 2 * d), lambda i: (i, 0)),
        out_shape=jax.ShapeDtypeStruct((nblocks * c // 2, 2 * d), jnp.float32),
    )(emb_t)


def _tc_pair_table_tail(tail_t):
    d, m = tail_t.shape

    def tr_kernel(x_ref, o_ref):
        t = jnp.swapaxes(x_ref[...], 0, 1)
        o_ref[:, :d] = t[: m // 2]
        o_ref[:, d:] = t[m // 2 :]

    return pl.pallas_call(
        tr_kernel,
        in_specs=[pl.BlockSpec((d, m), lambda: (0, 0))],
        out_specs=pl.BlockSpec((m // 2, 2 * d), lambda: (0, 0)),
        out_shape=jax.ShapeDtypeStruct((m // 2, 2 * d), jnp.float32),
    )(tail_t)


def _sc_gather(bulk, tail, idx_bulk, idx_tail):
    n = idx_bulk.shape[0]
    d = bulk.shape[1]
    ib2 = idx_bulk.reshape(1, n)
    it2 = idx_tail.reshape(1, n)
    mesh = plsc.VectorSubcoreMesh(core_axis_name="c", subcore_axis_name="s")
    w = _GATHER_WINDOW

    @functools.partial(
        pl.kernel,
        mesh=mesh,
        out_type=jax.ShapeDtypeStruct((n, d), bulk.dtype),
    )
    def gather_kernel(bulk_hbm, tail_hbm, ib_hbm, it_hbm, out_hbm):
        def body(ib_vmem, it_vmem, o_vmem):
            pltpu.sync_copy(
                bulk_hbm.at[plsc.Indices(ib_vmem.at[0], ignored_value=-1)],
                o_vmem,
            )
            pltpu.sync_copy(
                tail_hbm.at[plsc.Indices(it_vmem.at[0], ignored_value=-1)],
                o_vmem,
            )

        pltpu.emit_pipeline(
            body,
            grid=(n // w,),
            in_specs=[
                pl.BlockSpec((1, w), index_map=lambda i: (0, i)),
                pl.BlockSpec((1, w), index_map=lambda i: (0, i)),
            ],
            out_specs=[pl.BlockSpec((w, d), index_map=lambda i: (i, 0))],
            core_axis_name=("c", "s"),
            dimension_semantics=(pltpu.PARALLEL,),
        )(ib_hbm, it_hbm, out_hbm)

    return gather_kernel(bulk, tail, ib2, it2)


def _tc_mlp_ln(pairs, sel, W, b, gamma, beta):
    n = pairs.shape[0]
    d = W.shape[0]
    h = W.shape[1]
    b2 = b.reshape(1, h)
    g2 = gamma.reshape(1, h)
    be2 = beta.reshape(1, h)

    def mlp_kernel(pair_ref, sel_ref, w_ref, b_ref, g_ref, be_ref, o_ref):
        p = sel_ref[...].astype(jnp.float32)
        x = pair_ref[:, :d] * (1.0 - p) + pair_ref[:, d:] * p
        acc = jnp.dot(x, w_ref[...], preferred_element_type=jnp.float32)
        acc = jnp.maximum(acc + b_ref[...], 0.0)
        mean = jnp.mean(acc, axis=-1, keepdims=True)
        var = jnp.mean((acc - mean) ** 2, axis=-1, keepdims=True)
        xhat = (acc - mean) * jax.lax.rsqrt(var + _EPS)
        o_ref[...] = xhat * g_ref[...] + be_ref[...]

    blk = min(_TC_BLOCK, n)
    return pl.pallas_call(
        mlp_kernel,
        grid=(n // blk,),
        in_specs=[
            pl.BlockSpec((blk, 2 * d), lambda i: (i, 0)),
            pl.BlockSpec((blk, 1), lambda i: (i, 0)),
            pl.BlockSpec((d, h), lambda i: (0, 0)),
            pl.BlockSpec((1, h), lambda i: (0, 0)),
            pl.BlockSpec((1, h), lambda i: (0, 0)),
            pl.BlockSpec((1, h), lambda i: (0, 0)),
        ],
        out_specs=pl.BlockSpec((blk, h), lambda i: (i, 0)),
        out_shape=jax.ShapeDtypeStruct((n, h), jnp.float32),
    )(pairs, sel, W, b2, g2, be2)


def kernel(user_input, emb, W, b, gamma, beta):
    n_rows, d = emb.shape
    c = _TR_COLS
    nblocks = n_rows // c
    bulk_rows = nblocks * c
    emb_t = emb.T
    bulk_pt = _tc_pair_table_bulk(emb_t, nblocks)
    tail_pt = _tc_pair_table_tail(
        lax.slice(emb_t, (0, bulk_rows), (d, n_rows))
    )

    r = user_input
    tail = r >= bulk_rows
    off = r % c
    bulk_prow = (r // c) * (c // 2) + (off % (c // 2))
    bulk_sel = off // (c // 2)
    to = r - bulk_rows
    tail_half = (n_rows - bulk_rows) // 2
    tail_prow = to % tail_half
    tail_sel = to // tail_half
    sel = jnp.where(tail, tail_sel, bulk_sel).astype(jnp.int8).reshape(-1, 1)
    idx_bulk = jnp.where(tail, -1, bulk_prow).astype(jnp.int32)
    idx_tail = jnp.where(tail, tail_prow, -1).astype(jnp.int32)

    gathered = _sc_gather(bulk_pt, tail_pt, idx_bulk, idx_tail)
    return _tc_mlp_ln(gathered, sel, W, b, gamma, beta)
